# baseline (speedup 1.0000x reference)
import functools

import jax
import jax.numpy as jnp
from jax import lax
from jax.experimental import pallas as pl
from jax.experimental.pallas import tpu as pltpu
from jax.experimental.pallas import tpu_sc as plsc

N_ENT = 100000
D_IN = 128
D_OUT = 256
D_REL = 128
B = 16384

ROW_BLK = 20000
SUB_BLK = 2048

NC = 2
NS = 16
NW = NC * NS
BPW = B // NW
CH = 128
NCH = BPW // CH


def _mm_body(a_ref, w_ref, b_ref, o_ref):
    acc = jnp.dot(a_ref[...], w_ref[...], preferred_element_type=jnp.float32)
    o_ref[...] = jnp.tanh(acc + b_ref[...])


def _project(x, pca_weight, bias2d, row_blk, n=None):
    if n is None:
        n = x.shape[0]
    return pl.pallas_call(
        _mm_body,
        grid=(n // row_blk,),
        in_specs=[
            pl.BlockSpec((row_blk, D_IN), lambda i: (i, 0)),
            pl.BlockSpec((D_IN, D_OUT), lambda i: (0, 0)),
            pl.BlockSpec((1, D_OUT), lambda i: (0, 0)),
        ],
        out_specs=pl.BlockSpec((row_blk, D_OUT), lambda i: (i, 0)),
        out_shape=jax.ShapeDtypeStruct((n, D_OUT), jnp.float32),
    )(x, pca_weight, bias2d)


def _mm_ilv_body(a_ref, w_ref, b_ref, o_ref):
    h = jnp.tanh(jnp.dot(a_ref[...], w_ref[...],
                         preferred_element_type=jnp.float32) + b_ref[...])
    o_ref[...] = h.reshape(h.shape[0] * 2, D_OUT // 2)


def _project_ilv(x, pca_weight, bias2d, row_blk):
    n = x.shape[0]
    return pl.pallas_call(
        _mm_ilv_body,
        grid=(n // row_blk,),
        in_specs=[
            pl.BlockSpec((row_blk, D_IN), lambda i: (i, 0)),
            pl.BlockSpec((D_IN, D_OUT), lambda i: (0, 0)),
            pl.BlockSpec((1, D_OUT), lambda i: (0, 0)),
        ],
        out_specs=pl.BlockSpec((2 * row_blk, D_OUT // 2), lambda i: (i, 0)),
        out_shape=jax.ShapeDtypeStruct((2 * n, D_OUT // 2), jnp.float32),
        compiler_params=pltpu.CompilerParams(
            vmem_limit_bytes=100 * 1024 * 1024),
    )(x, pca_weight, bias2d)


_sc_mesh = plsc.VectorSubcoreMesh(core_axis_name="c", subcore_axis_name="s")


@functools.partial(
    pl.kernel,
    out_type=jax.ShapeDtypeStruct((2 * B, D_IN), jnp.float32),
    mesh=_sc_mesh,
    scratch_types=[
        pltpu.VMEM((BPW,), jnp.int32),
        pltpu.VMEM((BPW,), jnp.int32),
        pltpu.VMEM((BPW, D_IN), jnp.float32),
        pltpu.VMEM((2 * CH, D_REL), jnp.float32),
        pltpu.SemaphoreType.DMA,
        pltpu.SemaphoreType.DMA,
    ],
)
def _sc_gather_inputs(embt_hbm, sub_hbm, relt_hbm, rel_hbm,
                      out, sidx, ridx, gbuf, rbuf, gsem, rsem):
    wid = lax.axis_index("s") * NC + lax.axis_index("c")
    base = wid * BPW
    pltpu.sync_copy(sub_hbm.at[pl.ds(base, BPW)], sidx)
    pltpu.sync_copy(rel_hbm.at[pl.ds(base, BPW)], ridx)
    ge_cps = []
    for c in range(NCH):
        ge_cps.append(pltpu.async_copy(
            embt_hbm.at[sidx.at[pl.ds(c * CH, CH)]],
            gbuf.at[pl.ds(c * CH, CH)], gsem))
    for r in range(2):
        cps = []
        for c in range(2):
            cps.append(pltpu.async_copy(
                relt_hbm.at[ridx.at[pl.ds((2 * r + c) * CH, CH)]],
                rbuf.at[pl.ds(c * CH, CH)], rsem))
        for cp in cps:
            cp.wait()
        pltpu.sync_copy(rbuf, out.at[pl.ds(B + base + 2 * r * CH, 2 * CH)])
    for cp in ge_cps:
        cp.wait()
    pltpu.sync_copy(gbuf, out.at[pl.ds(base, BPW)])


def kernel(sub, rel, init_embed, init_rel, pca_weight, pca_bias):
    bias2d = pca_bias.reshape(1, D_OUT)
    gr = _sc_gather_inputs(init_embed, sub.astype(jnp.int32),
                           init_rel, rel.astype(jnp.int32))
    y = _project_ilv(init_embed, pca_weight, bias2d, ROW_BLK)
    sub_emb = _project(gr, pca_weight, bias2d, SUB_BLK, n=B)
    return sub_emb, gr[B:], y.reshape(N_ENT, 2, D_OUT // 2)


# speedup vs baseline: 1.0474x; 1.0474x over previous
import functools

import jax
import jax.numpy as jnp
from jax import lax
from jax.experimental import pallas as pl
from jax.experimental.pallas import tpu as pltpu
from jax.experimental.pallas import tpu_sc as plsc

N_ENT = 100000
D_IN = 128
D_OUT = 256
D_REL = 128
B = 16384

ROW_BLK = 20000
SUB_BLK = 2048

NC = 2
NS = 16
NW = NC * NS
BPW = B // NW
CH = 128
NCH = BPW // CH


def _mm_body(a_ref, w_ref, b_ref, o_ref):
    acc = jnp.dot(a_ref[...], w_ref[...], preferred_element_type=jnp.float32)
    o_ref[...] = jnp.tanh(acc + b_ref[...])


def _project(x, pca_weight, bias2d, row_blk, n=None):
    if n is None:
        n = x.shape[0]
    return pl.pallas_call(
        _mm_body,
        grid=(n // row_blk,),
        in_specs=[
            pl.BlockSpec((row_blk, D_IN), lambda i: (i, 0)),
            pl.BlockSpec((D_IN, D_OUT), lambda i: (0, 0)),
            pl.BlockSpec((1, D_OUT), lambda i: (0, 0)),
        ],
        out_specs=pl.BlockSpec((row_blk, D_OUT), lambda i: (i, 0)),
        out_shape=jax.ShapeDtypeStruct((n, D_OUT), jnp.float32),
    )(x, pca_weight, bias2d)


def _mm_ilv_body(a_ref, w_ref, b_ref, o_ref):
    h = jnp.tanh(jnp.dot(a_ref[...], w_ref[...],
                         preferred_element_type=jnp.float32) + b_ref[...])
    o_ref[...] = h.reshape(h.shape[0] * 2, D_OUT // 2)


def _project_ilv(x, pca_weight, bias2d, row_blk):
    n = x.shape[0]
    return pl.pallas_call(
        _mm_ilv_body,
        grid=(n // row_blk,),
        in_specs=[
            pl.BlockSpec((row_blk, D_IN), lambda i: (i, 0)),
            pl.BlockSpec((D_IN, D_OUT), lambda i: (0, 0)),
            pl.BlockSpec((1, D_OUT), lambda i: (0, 0)),
        ],
        out_specs=pl.BlockSpec((2 * row_blk, D_OUT // 2), lambda i: (i, 0)),
        out_shape=jax.ShapeDtypeStruct((2 * n, D_OUT // 2), jnp.float32),
        compiler_params=pltpu.CompilerParams(
            vmem_limit_bytes=100 * 1024 * 1024),
    )(x, pca_weight, bias2d)


_sc_mesh = plsc.VectorSubcoreMesh(core_axis_name="c", subcore_axis_name="s")


@functools.partial(
    pl.kernel,
    out_type=(jax.ShapeDtypeStruct((B, D_IN), jnp.float32),
              jax.ShapeDtypeStruct((B, D_REL), jnp.float32)),
    mesh=_sc_mesh,
    scratch_types=[
        pltpu.VMEM((BPW,), jnp.int32),
        pltpu.VMEM((BPW,), jnp.int32),
        pltpu.VMEM((BPW, D_IN), jnp.float32),
        pltpu.VMEM((2 * CH, D_REL), jnp.float32),
        pltpu.SemaphoreType.DMA,
        pltpu.SemaphoreType.DMA,
    ],
)
def _sc_gather_inputs(embt_hbm, sub_hbm, relt_hbm, rel_hbm,
                      ge_out, rel_out, sidx, ridx, gbuf, rbuf, gsem, rsem):
    wid = lax.axis_index("s") * NC + lax.axis_index("c")
    base = wid * BPW
    pltpu.sync_copy(sub_hbm.at[pl.ds(base, BPW)], sidx)
    pltpu.sync_copy(rel_hbm.at[pl.ds(base, BPW)], ridx)
    ge_cps = []
    for c in range(NCH):
        ge_cps.append(pltpu.async_copy(
            embt_hbm.at[sidx.at[pl.ds(c * CH, CH)]],
            gbuf.at[pl.ds(c * CH, CH)], gsem))
    for r in range(2):
        cps = []
        for c in range(2):
            cps.append(pltpu.async_copy(
                relt_hbm.at[ridx.at[pl.ds((2 * r + c) * CH, CH)]],
                rbuf.at[pl.ds(c * CH, CH)], rsem))
        for cp in cps:
            cp.wait()
        pltpu.sync_copy(rbuf, rel_out.at[pl.ds(base + 2 * r * CH, 2 * CH)])
    for cp in ge_cps:
        cp.wait()
    pltpu.sync_copy(gbuf, ge_out.at[pl.ds(base, BPW)])


def kernel(sub, rel, init_embed, init_rel, pca_weight, pca_bias):
    bias2d = pca_bias.reshape(1, D_OUT)
    ge, rel_emb = _sc_gather_inputs(init_embed, sub.astype(jnp.int32),
                                    init_rel, rel.astype(jnp.int32))
    y = _project_ilv(init_embed, pca_weight, bias2d, ROW_BLK)
    sub_emb = _project(ge, pca_weight, bias2d, SUB_BLK)
    return sub_emb, rel_emb, y.reshape(N_ENT, 2, D_OUT // 2)


# speedup vs baseline: 1.0626x; 1.0145x over previous
import functools

import jax
import jax.numpy as jnp
from jax import lax
from jax.experimental import pallas as pl
from jax.experimental.pallas import tpu as pltpu
from jax.experimental.pallas import tpu_sc as plsc

N_ENT = 100000
D_IN = 128
D_OUT = 256
D_REL = 128
B = 16384

ROW_BLK = 10000
SUB_BLK = 2048

NC = 2
NS = 16
NW = NC * NS
BPW = B // NW
CH = 128
NCH = BPW // CH


def _mm_body(a_ref, w_ref, b_ref, o_ref):
    acc = jnp.dot(a_ref[...], w_ref[...], preferred_element_type=jnp.float32)
    o_ref[...] = jnp.tanh(acc + b_ref[...])


def _project(x, pca_weight, bias2d, row_blk, n=None):
    if n is None:
        n = x.shape[0]
    return pl.pallas_call(
        _mm_body,
        grid=(n // row_blk,),
        in_specs=[
            pl.BlockSpec((row_blk, D_IN), lambda i: (i, 0)),
            pl.BlockSpec((D_IN, D_OUT), lambda i: (0, 0)),
            pl.BlockSpec((1, D_OUT), lambda i: (0, 0)),
        ],
        out_specs=pl.BlockSpec((row_blk, D_OUT), lambda i: (i, 0)),
        out_shape=jax.ShapeDtypeStruct((n, D_OUT), jnp.float32),
    )(x, pca_weight, bias2d)


def _mm_ilv_body(a_ref, w_ref, b_ref, o_ref):
    h = jnp.tanh(jnp.dot(a_ref[...], w_ref[...],
                         preferred_element_type=jnp.float32) + b_ref[...])
    o_ref[...] = h.reshape(h.shape[0] * 2, D_OUT // 2)


def _project_ilv(x, pca_weight, bias2d, row_blk):
    n = x.shape[0]
    return pl.pallas_call(
        _mm_ilv_body,
        grid=(n // row_blk,),
        in_specs=[
            pl.BlockSpec((row_blk, D_IN), lambda i: (i, 0)),
            pl.BlockSpec((D_IN, D_OUT), lambda i: (0, 0)),
            pl.BlockSpec((1, D_OUT), lambda i: (0, 0)),
        ],
        out_specs=pl.BlockSpec((2 * row_blk, D_OUT // 2), lambda i: (i, 0)),
        out_shape=jax.ShapeDtypeStruct((2 * n, D_OUT // 2), jnp.float32),
        compiler_params=pltpu.CompilerParams(
            vmem_limit_bytes=100 * 1024 * 1024),
    )(x, pca_weight, bias2d)


_sc_mesh = plsc.VectorSubcoreMesh(core_axis_name="c", subcore_axis_name="s")


@functools.partial(
    pl.kernel,
    out_type=(jax.ShapeDtypeStruct((B, D_IN), jnp.float32),
              jax.ShapeDtypeStruct((B, D_REL), jnp.float32)),
    mesh=_sc_mesh,
    scratch_types=[
        pltpu.VMEM((BPW,), jnp.int32),
        pltpu.VMEM((BPW,), jnp.int32),
        pltpu.VMEM((BPW, D_IN), jnp.float32),
        pltpu.VMEM((2 * CH, D_REL), jnp.float32),
        pltpu.SemaphoreType.DMA,
        pltpu.SemaphoreType.DMA,
    ],
)
def _sc_gather_inputs(embt_hbm, sub_hbm, relt_hbm, rel_hbm,
                      ge_out, rel_out, sidx, ridx, gbuf, rbuf, gsem, rsem):
    wid = lax.axis_index("s") * NC + lax.axis_index("c")
    base = wid * BPW
    pltpu.sync_copy(sub_hbm.at[pl.ds(base, BPW)], sidx)
    pltpu.sync_copy(rel_hbm.at[pl.ds(base, BPW)], ridx)
    ge_cps = []
    for c in range(NCH):
        ge_cps.append(pltpu.async_copy(
            embt_hbm.at[sidx.at[pl.ds(c * CH, CH)]],
            gbuf.at[pl.ds(c * CH, CH)], gsem))
    for r in range(2):
        cps = []
        for c in range(2):
            cps.append(pltpu.async_copy(
                relt_hbm.at[ridx.at[pl.ds((2 * r + c) * CH, CH)]],
                rbuf.at[pl.ds(c * CH, CH)], rsem))
        for cp in cps:
            cp.wait()
        pltpu.sync_copy(rbuf, rel_out.at[pl.ds(base + 2 * r * CH, 2 * CH)])
    for cp in ge_cps:
        cp.wait()
    pltpu.sync_copy(gbuf, ge_out.at[pl.ds(base, BPW)])


def kernel(sub, rel, init_embed, init_rel, pca_weight, pca_bias):
    bias2d = pca_bias.reshape(1, D_OUT)
    ge, rel_emb = _sc_gather_inputs(init_embed, sub.astype(jnp.int32),
                                    init_rel, rel.astype(jnp.int32))
    y = _project_ilv(init_embed, pca_weight, bias2d, ROW_BLK)
    sub_emb = _project(ge, pca_weight, bias2d, SUB_BLK)
    return sub_emb, rel_emb, y.reshape(N_ENT, 2, D_OUT // 2)
